# baseline (speedup 1.0000x reference)
import functools

import jax
import jax.numpy as jnp
from jax import lax
from jax.experimental import pallas as pl
from jax.experimental.pallas import tpu as pltpu
from jax.experimental.pallas import tpu_sc as plsc

B = 2
C = 32
N = 4096
K = 20
O = 64
R = 1024
NT = N // R

NW = 32
P = N // NW
CP = 32
NCH = P // CP
ROWS = CP * K
IDXW = 80
NDMA = ROWS // IDXW
OP = 128

_INV_SQRT = 0.9999950000374997


_NCH = 32
_KEEP = 8


def _ce(a, b):
    va, ia = a
    vb, ib = b
    m = vb < va
    return ((jnp.where(m, vb, va), jnp.where(m, ib, ia)),
            (jnp.where(m, va, vb), jnp.where(m, ia, ib)))


def _mn(a, b):
    va, ia = a
    vb, ib = b
    m = vb < va
    return (jnp.where(m, vb, va), jnp.where(m, ib, ia))


def _bitonic(seq):
    n = len(seq)
    if n == 1:
        return seq
    half = n // 2
    lo, hi = [], []
    for i in range(half):
        l, h = _ce(seq[i], seq[i + half])
        lo.append(l)
        hi.append(h)
    return _bitonic(lo) + _bitonic(hi)


def _merge(a_list, b_list):
    return _bitonic(a_list + b_list[::-1])


def _merge_keep(a_list, b_list, resort):
    m = len(a_list)
    mins = [_mn(a_list[i], b_list[m - 1 - i]) for i in range(m)]
    return _bitonic(mins) if resort else mins


_DN0 = (((0,), (0,)), ((), ()))


def _knn_tc_kernel(xt_ref, xf_ref, w1t_ref, wdt_ref, idx_ref, y_ref, z_ref):
    xtile = xt_ref[0]
    xfull = xf_ref[0]
    inner = lax.dot_general(xtile, xfull, _DN0,
                            preferred_element_type=jnp.float32)
    colsq = jnp.sum(xfull * xfull, axis=0, keepdims=True)
    s = colsq - 2.0 * inner

    lists = [[(s[:, c * 128:(c + 1) * 128],
               jnp.full((R, 128), c, jnp.int32))] for c in range(_NCH)]
    while len(lists) > 1:
        nxt = []
        last = len(lists) == 2
        for i in range(0, len(lists), 2):
            if len(lists[i]) < _KEEP:
                nxt.append(_merge(lists[i], lists[i + 1]))
            else:
                nxt.append(_merge_keep(lists[i], lists[i + 1], not last))
        lists = nxt
    cand = lists[0]

    vv = jnp.concatenate([v for v, _ in cand], axis=1)
    ii = jnp.concatenate([i for _, i in cand], axis=1)
    lane = lax.broadcasted_iota(jnp.int32, (R, _KEEP * 128), 1) & 127
    g = ii * 128 + lane

    rowid = lax.broadcasted_iota(jnp.int32, (R, 1), 0) + pl.program_id(0) * R
    picks = [rowid]
    vv = jnp.where(g == rowid, jnp.float32(jnp.inf), vv)
    big = jnp.int32(N)
    for t in range(K - 1):
        m = jnp.min(vv, axis=1, keepdims=True)
        c = jnp.where(vv <= m, g, big)
        at = jnp.min(c, axis=1, keepdims=True)
        picks.append(at)
        if t + 2 < K:
            vv = jnp.where(c == at, jnp.float32(jnp.inf), vv)
    idx_ref[...] = jnp.concatenate(picks, axis=1)
    y_ref[...] = lax.dot_general(xtile, w1t_ref[...], _DN0,
                                 preferred_element_type=jnp.float32)
    z_ref[...] = lax.dot_general(xtile, wdt_ref[...], _DN0,
                                 preferred_element_type=jnp.float32)


def _gather_max_sc_kernel(y_hbm, idx_hbm, z_hbm, gam_hbm, bet_hbm, out_hbm,
                          idx_v, rows_v, z_v, out_v, gb_v, sem):
    wid = lax.axis_index("s") * 2 + lax.axis_index("c")
    pltpu.sync_copy(idx_hbm.at[pl.ds(wid * (NCH * NDMA), NCH * NDMA)], idx_v)
    pltpu.sync_copy(z_hbm.at[pl.ds(wid * (P * O), P * O)], z_v)
    pltpu.sync_copy(gam_hbm, gb_v.at[pl.ds(0, O)])
    pltpu.sync_copy(bet_hbm, gb_v.at[pl.ds(O, O)])

    scale = [gb_v[pl.ds(16 * q, 16)] * _INV_SQRT for q in range(4)]
    bet = [gb_v[pl.ds(O + 16 * q, 16)] for q in range(4)]
    pos = [sc >= 0.0 for sc in scale]

    for ch in range(NCH):
        cps = [
            pltpu.async_copy(
                y_hbm.at[idx_v.at[ch * NDMA + i]],
                rows_v.at[pl.ds(i * IDXW, IDXW)],
                sem,
            )
            for i in range(NDMA)
        ]
        for cp in cps:
            cp.wait()

        def body(p, _, ch=ch):
            r0 = p * K
            mx = [rows_v[r0, pl.ds(16 * q, 16)] for q in range(4)]
            mn = list(mx)
            for j in range(1, K):
                for q in range(4):
                    v = rows_v[r0 + j, pl.ds(16 * q, 16)]
                    mx[q] = jnp.maximum(mx[q], v)
                    mn[q] = jnp.minimum(mn[q], v)
            gp = ch * CP + p
            for q in range(4):
                zrow = z_v[pl.ds(gp * O + 16 * q, 16)]
                chosen = jnp.where(pos[q], mx[q], mn[q])
                o = (chosen + zrow) * scale[q] + bet[q]
                o = jnp.where(o > 0.0, o, 0.2 * o)
                out_v[pl.ds(gp * O + 16 * q, 16)] = o
            return _

        lax.fori_loop(0, CP, body, 0, unroll=False)

    pltpu.sync_copy(out_v, out_hbm.at[pl.ds(wid * (P * O), P * O)])


@jax.jit
def kernel(x, W, gamma, beta):
    w1t = jnp.pad(jnp.transpose(W[:, :C]), ((0, 0), (0, OP - O)))
    wdt = jnp.transpose(W[:, C:] - W[:, :C])

    sc = functools.partial(
        pl.kernel,
        out_type=jax.ShapeDtypeStruct((N * O,), jnp.float32),
        mesh=plsc.VectorSubcoreMesh(core_axis_name="c", subcore_axis_name="s"),
        scratch_types=[
            pltpu.VMEM((NCH * NDMA, IDXW), jnp.int32),
            pltpu.VMEM((ROWS, OP), jnp.float32),
            pltpu.VMEM((P * O,), jnp.float32),
            pltpu.VMEM((P * O,), jnp.float32),
            pltpu.VMEM((2 * O,), jnp.float32),
            pltpu.SemaphoreType.DMA,
        ],
    )(_gather_max_sc_kernel)

    ms = []
    for b in range(B):
        idx, y, z = pl.pallas_call(
            _knn_tc_kernel,
            grid=(NT,),
            in_specs=[
                pl.BlockSpec((1, C, R), lambda r, b_=b: (b_, 0, r)),
                pl.BlockSpec((1, C, N), lambda r, b_=b: (b_, 0, 0)),
                pl.BlockSpec((C, OP), lambda r: (0, 0)),
                pl.BlockSpec((C, O), lambda r: (0, 0)),
            ],
            out_specs=[
                pl.BlockSpec((R, K), lambda r: (r, 0)),
                pl.BlockSpec((R, OP), lambda r: (r, 0)),
                pl.BlockSpec((R, O), lambda r: (r, 0)),
            ],
            out_shape=[
                jax.ShapeDtypeStruct((N, K), jnp.int32),
                jax.ShapeDtypeStruct((N, OP), jnp.float32),
                jax.ShapeDtypeStruct((N, O), jnp.float32),
            ],
        )(x, x, w1t, wdt)
        m = sc(y, idx.reshape(NW * NCH * NDMA, IDXW), z.reshape(N * O),
               gamma, beta)
        ms.append(m.reshape(N, O))
    return jnp.transpose(jnp.stack(ms), (0, 2, 1))


# speedup vs baseline: 1.3238x; 1.3238x over previous
import functools

import jax
import jax.numpy as jnp
from jax import lax
from jax.experimental import pallas as pl
from jax.experimental.pallas import tpu as pltpu
from jax.experimental.pallas import tpu_sc as plsc

B = 2
C = 32
N = 4096
K = 20
O = 64
R = 512
NT = N // R

NW = 32
P = N // NW
CP = 32
NCH = P // CP
ROWS = CP * K
IDXW = 80
NDMA = ROWS // IDXW
OP = 128

_INV_SQRT = 0.9999950000374997


_NCH = 32
_KEEP = 8


def _ce(a, b):
    va, ia = a
    vb, ib = b
    m = vb < va
    return ((jnp.where(m, vb, va), jnp.where(m, ib, ia)),
            (jnp.where(m, va, vb), jnp.where(m, ia, ib)))


def _mn(a, b):
    va, ia = a
    vb, ib = b
    m = vb < va
    return (jnp.where(m, vb, va), jnp.where(m, ib, ia))


def _bitonic(seq):
    n = len(seq)
    if n == 1:
        return seq
    half = n // 2
    lo, hi = [], []
    for i in range(half):
        l, h = _ce(seq[i], seq[i + half])
        lo.append(l)
        hi.append(h)
    return _bitonic(lo) + _bitonic(hi)


def _merge(a_list, b_list):
    return _bitonic(a_list + b_list[::-1])


def _merge_keep(a_list, b_list, resort):
    m = len(a_list)
    mins = [_mn(a_list[i], b_list[m - 1 - i]) for i in range(m)]
    return _bitonic(mins) if resort else mins


_DN0 = (((0,), (0,)), ((), ()))


def _knn_tc_kernel(xt_ref, xf_ref, w1t_ref, wdt_ref, idx_ref, y_ref, z_ref):
    xtile = xt_ref[0]
    xfull = xf_ref[0]
    inner = lax.dot_general(xtile, xfull, _DN0,
                            preferred_element_type=jnp.float32)
    colsq = jnp.sum(xfull * xfull, axis=0, keepdims=True)
    s = colsq - 2.0 * inner

    lists = [[(s[:, c * 128:(c + 1) * 128],
               jnp.full((R, 128), c, jnp.int32))] for c in range(_NCH)]
    while len(lists) > 1:
        nxt = []
        last = len(lists) == 2
        for i in range(0, len(lists), 2):
            if len(lists[i]) < _KEEP:
                nxt.append(_merge(lists[i], lists[i + 1]))
            else:
                nxt.append(_merge_keep(lists[i], lists[i + 1], not last))
        lists = nxt
    cand = lists[0]

    vv = jnp.concatenate([v for v, _ in cand], axis=1)
    ii = jnp.concatenate([i for _, i in cand], axis=1)
    lane = lax.broadcasted_iota(jnp.int32, (R, _KEEP * 128), 1) & 127
    g = ii * 128 + lane

    rowid = lax.broadcasted_iota(jnp.int32, (R, 1), 0) + pl.program_id(0) * R
    picks = [rowid]
    vv = jnp.where(g == rowid, jnp.float32(jnp.inf), vv)
    big = jnp.int32(N)
    for t in range(K - 1):
        m = jnp.min(vv, axis=1, keepdims=True)
        c = jnp.where(vv <= m, g, big)
        at = jnp.min(c, axis=1, keepdims=True)
        picks.append(at)
        if t + 2 < K:
            vv = jnp.where(c == at, jnp.float32(jnp.inf), vv)
    idx_ref[...] = jnp.concatenate(picks, axis=1)
    y_ref[...] = lax.dot_general(xtile, w1t_ref[...], _DN0,
                                 preferred_element_type=jnp.float32)
    z_ref[...] = lax.dot_general(xtile, wdt_ref[...], _DN0,
                                 preferred_element_type=jnp.float32)


def _gather_max_sc_kernel(y_hbm, idx_hbm, z_hbm, gam_hbm, bet_hbm, out_hbm,
                          idx_v, rows_v, z_v, out_v, gb_v, sem):
    wid = lax.axis_index("s") * 2 + lax.axis_index("c")
    pltpu.sync_copy(idx_hbm.at[pl.ds(wid * (NCH * NDMA), NCH * NDMA)], idx_v)
    pltpu.sync_copy(z_hbm.at[pl.ds(wid * (P * O), P * O)], z_v)
    pltpu.sync_copy(gam_hbm, gb_v.at[pl.ds(0, O)])
    pltpu.sync_copy(bet_hbm, gb_v.at[pl.ds(O, O)])

    scale = [gb_v[pl.ds(16 * q, 16)] * _INV_SQRT for q in range(4)]
    bet = [gb_v[pl.ds(O + 16 * q, 16)] for q in range(4)]
    pos = [sc >= 0.0 for sc in scale]

    for ch in range(NCH):
        cps = [
            pltpu.async_copy(
                y_hbm.at[idx_v.at[ch * NDMA + i]],
                rows_v.at[pl.ds(i * IDXW, IDXW)],
                sem,
            )
            for i in range(NDMA)
        ]
        for cp in cps:
            cp.wait()

        def body(p, _, ch=ch):
            r0 = p * K
            mx = [rows_v[r0, pl.ds(16 * q, 16)] for q in range(4)]
            mn = list(mx)
            for j in range(1, K):
                for q in range(4):
                    v = rows_v[r0 + j, pl.ds(16 * q, 16)]
                    mx[q] = jnp.maximum(mx[q], v)
                    mn[q] = jnp.minimum(mn[q], v)
            gp = ch * CP + p
            for q in range(4):
                zrow = z_v[pl.ds(gp * O + 16 * q, 16)]
                chosen = jnp.where(pos[q], mx[q], mn[q])
                o = (chosen + zrow) * scale[q] + bet[q]
                o = jnp.where(o > 0.0, o, 0.2 * o)
                out_v[pl.ds(gp * O + 16 * q, 16)] = o
            return _

        lax.fori_loop(0, CP, body, 0, unroll=False)

    pltpu.sync_copy(out_v, out_hbm.at[pl.ds(wid * (P * O), P * O)])


@jax.jit
def kernel(x, W, gamma, beta):
    w1t = jnp.pad(jnp.transpose(W[:, :C]), ((0, 0), (0, OP - O)))
    wdt = jnp.transpose(W[:, C:] - W[:, :C])

    sc = functools.partial(
        pl.kernel,
        out_type=jax.ShapeDtypeStruct((N * O,), jnp.float32),
        mesh=plsc.VectorSubcoreMesh(core_axis_name="c", subcore_axis_name="s"),
        scratch_types=[
            pltpu.VMEM((NCH * NDMA, IDXW), jnp.int32),
            pltpu.VMEM((ROWS, OP), jnp.float32),
            pltpu.VMEM((P * O,), jnp.float32),
            pltpu.VMEM((P * O,), jnp.float32),
            pltpu.VMEM((2 * O,), jnp.float32),
            pltpu.SemaphoreType.DMA,
        ],
    )(_gather_max_sc_kernel)

    ms = []
    for b in range(B):
        idx, y, z = pl.pallas_call(
            _knn_tc_kernel,
            grid=(NT,),
            in_specs=[
                pl.BlockSpec((1, C, R), lambda r, b_=b: (b_, 0, r)),
                pl.BlockSpec((1, C, N), lambda r, b_=b: (b_, 0, 0)),
                pl.BlockSpec((C, OP), lambda r: (0, 0)),
                pl.BlockSpec((C, O), lambda r: (0, 0)),
            ],
            out_specs=[
                pl.BlockSpec((R, K), lambda r: (r, 0)),
                pl.BlockSpec((R, OP), lambda r: (r, 0)),
                pl.BlockSpec((R, O), lambda r: (r, 0)),
            ],
            out_shape=[
                jax.ShapeDtypeStruct((N, K), jnp.int32),
                jax.ShapeDtypeStruct((N, OP), jnp.float32),
                jax.ShapeDtypeStruct((N, O), jnp.float32),
            ],
        )(x, x, w1t, wdt)
        m = sc(y, idx.reshape(NW * NCH * NDMA, IDXW), z.reshape(N * O),
               gamma, beta)
        ms.append(m.reshape(N, O))
    return jnp.transpose(jnp.stack(ms), (0, 2, 1))
